# baseline (speedup 1.0000x reference)
import functools

import jax
import jax.numpy as jnp
from jax import lax
from jax.experimental import pallas as pl
from jax.experimental.pallas import tpu as pltpu
from jax.experimental.pallas import tpu_sc as plsc

B = 16384
NSPLIT = 1
BSUB = B // NSPLIT
D = 32
NCAT = 1000
NC, NS = 2, 16
NW = NC * NS
ROWS_PER_W = BSUB // NW
CHUNK = 1
NBUF = 16
HALF = ROWS_PER_W // 2
HCHUNK = HALF // CHUNK


def _gather_body(ids_hbm, ut_hbm, it_hbm, uout_hbm, iout_hbm,
                 ids_v, buf, stage, *sems):
    wid = lax.axis_index("s") * NC + lax.axis_index("c")
    base = wid * ROWS_PER_W
    lane = lax.iota(jnp.int32, 16)
    zeros16 = lane * 0.0

    @pl.loop(0, HALF)
    def _zero(k):
        for cc in range(2, 8):
            stage[k, pl.ds(cc * 16, 16)] = zeros16

    for t, (src, outref) in enumerate(((ut_hbm, uout_hbm), (it_hbm, iout_hbm))):
        pltpu.sync_copy(ids_hbm.at[pl.ds(t * BSUB + base, ROWS_PER_W)],
                        ids_v.at[pl.ds(0, ROWS_PER_W)])

        def fire(c, slot):
            v = ids_v[pl.ds(c * CHUNK, 16)]
            for kk in range(CHUNK):
                col0 = pl.multiple_of((v[kk] >> 7) * 128, 128)
                pltpu.async_copy(
                    src.at[:, pl.ds(col0, 128)], buf.at[slot * CHUNK + kk],
                    sems[slot])

        def drain(slot):
            for kk in range(CHUNK):
                pltpu.make_async_copy(
                    src.at[:, pl.ds(0, 128)], buf.at[slot * CHUNK + kk],
                    sems[slot]).wait()

        def extract(c, k0, slot):
            v = ids_v[pl.ds(c * CHUNK, 16)]
            for kk in range(CHUNK):
                j = lane * 0 + (v[kk] & 127)
                c_b = lane * 0 + (slot * CHUNK + kk)
                stage[k0 + kk, pl.ds(0, 16)] = plsc.load_gather(
                    buf, [c_b, lane, j])
                stage[k0 + kk, pl.ds(16, 16)] = plsc.load_gather(
                    buf, [c_b, lane + 16, j])

        for h in range(2):
            c0 = h * HCHUNK
            for u in range(NBUF):
                fire(c0 + u, u)

            @pl.loop(0, HCHUNK, step=NBUF)
            def _round(g):
                for u in range(NBUF):
                    drain(u)
                    extract(c0 + g + u, (g + u) * CHUNK, u)
                    nxt = g + u + NBUF

                    @pl.when(nxt < HCHUNK)
                    def _():
                        fire(c0 + nxt, u)

            pltpu.sync_copy(stage, outref.at[pl.ds(base + h * HALF, HALF)])


@functools.cache
def _gather_kernel():
    return pl.kernel(
        _gather_body,
        out_type=(
            jax.ShapeDtypeStruct((BSUB, 128), jnp.float32),
            jax.ShapeDtypeStruct((BSUB, 128), jnp.float32),
        ),
        mesh=plsc.VectorSubcoreMesh(core_axis_name="c", subcore_axis_name="s"),
        compiler_params=pltpu.CompilerParams(needs_layout_passes=False),
        scratch_types=[
            pltpu.VMEM((ROWS_PER_W + 16,), jnp.int32),
            pltpu.VMEM((NBUF * CHUNK, D, 128), jnp.float32),
            pltpu.VMEM((HALF, 128), jnp.float32),
            pltpu.SemaphoreType.DMA,
            pltpu.SemaphoreType.DMA,
            pltpu.SemaphoreType.DMA,
            pltpu.SemaphoreType.DMA,
            pltpu.SemaphoreType.DMA,
            pltpu.SemaphoreType.DMA,
            pltpu.SemaphoreType.DMA,
            pltpu.SemaphoreType.DMA,
            pltpu.SemaphoreType.DMA,
            pltpu.SemaphoreType.DMA,
            pltpu.SemaphoreType.DMA,
            pltpu.SemaphoreType.DMA,
            pltpu.SemaphoreType.DMA,
            pltpu.SemaphoreType.DMA,
            pltpu.SemaphoreType.DMA,
            pltpu.SemaphoreType.DMA,
        ],
    )


_MLP_BLK = 2048


def _mlp_body(u_ref, i_ref, cid_ref, dense_ref, w1u_ref, w1i_ref, ct_ref,
              w1c_ref, w1d_ref, b1_ref, w2_ref, b2_ref, out_ref):
    d = dense_ref[...]
    h = jnp.dot(u_ref[...], w1u_ref[...], preferred_element_type=jnp.float32)
    h = h + jnp.dot(i_ref[...], w1i_ref[...], preferred_element_type=jnp.float32)
    m = lax.dot_general(ct_ref[...], w1c_ref[...], (((0,), (0,)), ((), ())),
                        preferred_element_type=jnp.float32)
    cid = cid_ref[...]
    cat_iota = lax.broadcasted_iota(jnp.int32, (_MLP_BLK, NCAT), 1)
    onehot = (cid == cat_iota).astype(jnp.float32)
    h = h + jnp.dot(onehot, m, preferred_element_type=jnp.float32)
    h = h + d[:, 0:1] * w1d_ref[0:1, :] + d[:, 1:2] * w1d_ref[1:2, :]
    h = jnp.maximum(h + b1_ref[...], 0.0)
    o = jnp.sum(h * w2_ref[...], axis=1, keepdims=True) + b2_ref[...]
    out_ref[...] = 1.0 / (1.0 + jnp.exp(-o))


def _tc_mlp(u, i, cid, dense, w1u, w1i, ct_t, w1c, w1d, b1r, w2r, b2r):
    grid = (BSUB // _MLP_BLK,)
    wide_spec = pl.BlockSpec((_MLP_BLK, 128), lambda i: (i, 0))
    w_spec = pl.BlockSpec((128, 16), lambda i: (0, 0))
    return pl.pallas_call(
        _mlp_body,
        grid=grid,
        in_specs=[
            wide_spec, wide_spec,
            pl.BlockSpec((_MLP_BLK, 1), lambda i: (i, 0)),
            pl.BlockSpec((_MLP_BLK, 2), lambda i: (i, 0)),
            w_spec, w_spec,
            pl.BlockSpec((D, NCAT), lambda i: (0, 0)),
            pl.BlockSpec((D, 16), lambda i: (0, 0)),
            pl.BlockSpec((2, 16), lambda i: (0, 0)),
            pl.BlockSpec((1, 16), lambda i: (0, 0)),
            pl.BlockSpec((1, 16), lambda i: (0, 0)),
            pl.BlockSpec((1, 1), lambda i: (0, 0)),
        ],
        out_specs=pl.BlockSpec((_MLP_BLK, 1), lambda i: (i, 0)),
        out_shape=jax.ShapeDtypeStruct((BSUB, 1), jnp.float32),
    )(u, i, cid, dense, w1u, w1i, ct_t, w1c, w1d, b1r, w2r, b2r)


def kernel(user_id, item_id, category_id, dense, user_table, item_table,
           cat_table, W1, b1, W2, b2):
    uid = user_id.astype(jnp.int32)
    iid = item_id.astype(jnp.int32)
    cid = category_id.astype(jnp.int32).reshape(B, 1)
    pad = jnp.zeros((128 - D, 16), jnp.float32)
    w1u = jnp.concatenate([W1[:D], pad])
    w1i = jnp.concatenate([W1[D:2 * D], pad])
    args = (cat_table.T, W1[2 * D:3 * D], W1[3 * D:],
            b1.reshape(1, 16), W2.reshape(1, 16), b2.reshape(1, 1))
    gathered = []
    for k in range(NSPLIT):
        sl = slice(k * BSUB, (k + 1) * BSUB)
        ids2 = jnp.concatenate([uid[sl], iid[sl]])
        gathered.append(_gather_kernel()(ids2, user_table.T, item_table.T))
    outs = []
    for k in range(NSPLIT):
        sl = slice(k * BSUB, (k + 1) * BSUB)
        u, i = gathered[k]
        outs.append(_tc_mlp(u, i, cid[sl], dense[sl], w1u, w1i, *args))
    return jnp.concatenate(outs)


# speedup vs baseline: 1.0843x; 1.0843x over previous
import functools

import jax
import jax.numpy as jnp
from jax import lax
from jax.experimental import pallas as pl
from jax.experimental.pallas import tpu as pltpu
from jax.experimental.pallas import tpu_sc as plsc

B = 16384
NSPLIT = 1
BSUB = B // NSPLIT
D = 32
NCAT = 1000
NC, NS = 2, 16
NW = NC * NS
ROWS_PER_W = BSUB // NW
CHUNK = 2
NBUF = 8
HALF = ROWS_PER_W // 2
HCHUNK = HALF // CHUNK


def _gather_body(ids_hbm, ut_hbm, it_hbm, uout_hbm, iout_hbm,
                 ids_v, buf, stage, *sems):
    wid = lax.axis_index("s") * NC + lax.axis_index("c")
    base = wid * ROWS_PER_W
    lane = lax.iota(jnp.int32, 16)
    zeros16 = lane * 0.0

    @pl.loop(0, HALF)
    def _zero(k):
        for cc in range(2, 8):
            stage[k, pl.ds(cc * 16, 16)] = zeros16

    for t, (src, outref) in enumerate(((ut_hbm, uout_hbm), (it_hbm, iout_hbm))):
        pltpu.sync_copy(ids_hbm.at[pl.ds(t * BSUB + base, ROWS_PER_W)],
                        ids_v.at[pl.ds(0, ROWS_PER_W)])

        def fire(c, slot):
            v = ids_v[pl.ds(c * CHUNK, 16)]
            for kk in range(CHUNK):
                col0 = pl.multiple_of((v[kk] >> 7) * 128, 128)
                pltpu.async_copy(
                    src.at[:, pl.ds(col0, 128)], buf.at[slot * CHUNK + kk],
                    sems[slot])

        def drain(slot):
            for kk in range(CHUNK):
                pltpu.make_async_copy(
                    src.at[:, pl.ds(0, 128)], buf.at[slot * CHUNK + kk],
                    sems[slot]).wait()

        def extract(c, k0, slot):
            v = ids_v[pl.ds(c * CHUNK, 16)]
            for kk in range(CHUNK):
                j = lane * 0 + (v[kk] & 127)
                c_b = lane * 0 + (slot * CHUNK + kk)
                stage[k0 + kk, pl.ds(0, 16)] = plsc.load_gather(
                    buf, [c_b, lane, j])
                stage[k0 + kk, pl.ds(16, 16)] = plsc.load_gather(
                    buf, [c_b, lane + 16, j])

        for h in range(2):
            c0 = h * HCHUNK
            for u in range(NBUF):
                fire(c0 + u, u)

            @pl.loop(0, HCHUNK, step=NBUF)
            def _round(g):
                for u in range(NBUF):
                    drain(u)
                    extract(c0 + g + u, (g + u) * CHUNK, u)
                    nxt = g + u + NBUF

                    @pl.when(nxt < HCHUNK)
                    def _():
                        fire(c0 + nxt, u)

            pltpu.sync_copy(stage, outref.at[pl.ds(base + h * HALF, HALF)])


@functools.cache
def _gather_kernel():
    return pl.kernel(
        _gather_body,
        out_type=(
            jax.ShapeDtypeStruct((BSUB, 128), jnp.float32),
            jax.ShapeDtypeStruct((BSUB, 128), jnp.float32),
        ),
        mesh=plsc.VectorSubcoreMesh(core_axis_name="c", subcore_axis_name="s"),
        compiler_params=pltpu.CompilerParams(needs_layout_passes=False),
        scratch_types=[
            pltpu.VMEM((ROWS_PER_W + 16,), jnp.int32),
            pltpu.VMEM((NBUF * CHUNK, D, 128), jnp.float32),
            pltpu.VMEM((HALF, 128), jnp.float32),
            pltpu.SemaphoreType.DMA,
            pltpu.SemaphoreType.DMA,
            pltpu.SemaphoreType.DMA,
            pltpu.SemaphoreType.DMA,
            pltpu.SemaphoreType.DMA,
            pltpu.SemaphoreType.DMA,
            pltpu.SemaphoreType.DMA,
            pltpu.SemaphoreType.DMA,
        ],
    )


_MLP_BLK = 2048


def _mlp_body(u_ref, i_ref, cid_ref, dense_ref, w1u_ref, w1i_ref, ct_ref,
              w1c_ref, w1d_ref, b1_ref, w2_ref, b2_ref, out_ref):
    d = dense_ref[...]
    h = jnp.dot(u_ref[...], w1u_ref[...], preferred_element_type=jnp.float32)
    h = h + jnp.dot(i_ref[...], w1i_ref[...], preferred_element_type=jnp.float32)
    m = lax.dot_general(ct_ref[...], w1c_ref[...], (((0,), (0,)), ((), ())),
                        preferred_element_type=jnp.float32)
    cid = cid_ref[...]
    cat_iota = lax.broadcasted_iota(jnp.int32, (_MLP_BLK, NCAT), 1)
    onehot = (cid == cat_iota).astype(jnp.float32)
    h = h + jnp.dot(onehot, m, preferred_element_type=jnp.float32)
    h = h + d[:, 0:1] * w1d_ref[0:1, :] + d[:, 1:2] * w1d_ref[1:2, :]
    h = jnp.maximum(h + b1_ref[...], 0.0)
    o = jnp.sum(h * w2_ref[...], axis=1, keepdims=True) + b2_ref[...]
    out_ref[...] = 1.0 / (1.0 + jnp.exp(-o))


def _tc_mlp(u, i, cid, dense, w1u, w1i, ct_t, w1c, w1d, b1r, w2r, b2r):
    grid = (BSUB // _MLP_BLK,)
    wide_spec = pl.BlockSpec((_MLP_BLK, 128), lambda i: (i, 0))
    w_spec = pl.BlockSpec((128, 16), lambda i: (0, 0))
    return pl.pallas_call(
        _mlp_body,
        grid=grid,
        in_specs=[
            wide_spec, wide_spec,
            pl.BlockSpec((_MLP_BLK, 1), lambda i: (i, 0)),
            pl.BlockSpec((_MLP_BLK, 2), lambda i: (i, 0)),
            w_spec, w_spec,
            pl.BlockSpec((D, NCAT), lambda i: (0, 0)),
            pl.BlockSpec((D, 16), lambda i: (0, 0)),
            pl.BlockSpec((2, 16), lambda i: (0, 0)),
            pl.BlockSpec((1, 16), lambda i: (0, 0)),
            pl.BlockSpec((1, 16), lambda i: (0, 0)),
            pl.BlockSpec((1, 1), lambda i: (0, 0)),
        ],
        out_specs=pl.BlockSpec((_MLP_BLK, 1), lambda i: (i, 0)),
        out_shape=jax.ShapeDtypeStruct((BSUB, 1), jnp.float32),
    )(u, i, cid, dense, w1u, w1i, ct_t, w1c, w1d, b1r, w2r, b2r)


def kernel(user_id, item_id, category_id, dense, user_table, item_table,
           cat_table, W1, b1, W2, b2):
    uid = user_id.astype(jnp.int32)
    iid = item_id.astype(jnp.int32)
    cid = category_id.astype(jnp.int32).reshape(B, 1)
    pad = jnp.zeros((128 - D, 16), jnp.float32)
    w1u = jnp.concatenate([W1[:D], pad])
    w1i = jnp.concatenate([W1[D:2 * D], pad])
    args = (cat_table.T, W1[2 * D:3 * D], W1[3 * D:],
            b1.reshape(1, 16), W2.reshape(1, 16), b2.reshape(1, 1))
    gathered = []
    for k in range(NSPLIT):
        sl = slice(k * BSUB, (k + 1) * BSUB)
        ids2 = jnp.concatenate([uid[sl], iid[sl]])
        gathered.append(_gather_kernel()(ids2, user_table.T, item_table.T))
    outs = []
    for k in range(NSPLIT):
        sl = slice(k * BSUB, (k + 1) * BSUB)
        u, i = gathered[k]
        outs.append(_tc_mlp(u, i, cid[sl], dense[sl], w1u, w1i, *args))
    return jnp.concatenate(outs)
